# Initial kernel scaffold; baseline (speedup 1.0000x reference)
import jax
import jax.numpy as jnp
from jax.experimental import pallas as pl


def kernel(nodes, neighbors, num_sample, features):
    raise NotImplementedError("write your pallas kernel here")


# speedup vs baseline: 1.3867x; 1.3867x over previous
import functools

import jax
import jax.numpy as jnp
from jax import lax
from jax.experimental import pallas as pl
from jax.experimental.pallas import tpu as pltpu
from jax.experimental.pallas import tpu_sc as plsc

L = 16


def kernel(nodes, neighbors, num_sample, features):
    B, S = neighbors.shape
    N, D = features.shape
    KD = D // L

    info = plsc.get_sparse_core_info()
    NC, NS = info.num_cores, info.num_subcores
    NW = NC * NS

    G = 128 // S
    CH = G * S

    rows_per_w = -(-B // NW)
    rows_per_w = -(-rows_per_w // (2 * G)) * (2 * G)
    nchunk = rows_per_w // G
    b_pad = rows_per_w * NW

    nb = neighbors.astype(jnp.int32)
    if b_pad != B:
        nb = jnp.concatenate(
            [nb, jnp.zeros((b_pad - B, S), jnp.int32)], axis=0)
    nb = nb.reshape(NW, nchunk, CH)

    mesh = plsc.VectorSubcoreMesh(core_axis_name="c", subcore_axis_name="s")

    @functools.partial(
        pl.kernel,
        mesh=mesh,
        out_type=jax.ShapeDtypeStruct((b_pad, D), jnp.float32),
        scratch_types=[
            pltpu.VMEM((nchunk, CH), jnp.int32),
            pltpu.VMEM((CH, D), jnp.float32),
            pltpu.VMEM((CH, D), jnp.float32),
            pltpu.VMEM((rows_per_w, D), jnp.float32),
            pltpu.SemaphoreType.DMA,
            pltpu.SemaphoreType.DMA,
        ],
    )
    def sc_kernel(nb_hbm, feat_hbm, out_hbm, idx_v, rows0, rows1, out_v,
                  sem0, sem1):
        wid = lax.axis_index("s") * NC + lax.axis_index("c")
        pltpu.sync_copy(nb_hbm.at[wid], idx_v)

        def start(g, rows, sem):
            pltpu.async_copy(feat_hbm.at[idx_v.at[g]], rows, sem)

        def wait(rows, sem):
            pltpu.make_async_copy(feat_hbm.at[idx_v.at[0]], rows, sem).wait()

        def compute(g, rows_ref):
            def do_row(r, carry):
                b0 = r * S
                accs = tuple(rows_ref[b0, pl.ds(k * L, L)] for k in range(KD))

                def s_body(s, accs):
                    return tuple(
                        jnp.maximum(a, rows_ref[b0 + s, pl.ds(k * L, L)])
                        for k, a in enumerate(accs))

                accs = lax.fori_loop(1, S, s_body, accs)
                row = g * G + r
                for k in range(KD):
                    out_v[row, pl.ds(k * L, L)] = accs[k]
                return carry

            lax.fori_loop(0, G, do_row, 0)

        start(0, rows0, sem0)
        start(1, rows1, sem1)

        def pair_body(i, carry):
            g = 2 * i
            wait(rows0, sem0)
            compute(g, rows0)

            @pl.when(g + 2 < nchunk)
            def _():
                start(g + 2, rows0, sem0)

            wait(rows1, sem1)
            compute(g + 1, rows1)

            @pl.when(g + 3 < nchunk)
            def _():
                start(g + 3, rows1, sem1)

            return carry

        lax.fori_loop(0, nchunk // 2, pair_body, 0)
        pltpu.sync_copy(out_v, out_hbm.at[pl.ds(wid * rows_per_w, rows_per_w)])

    out = sc_kernel(nb, features)
    return out[:B]


# speedup vs baseline: 7.4288x; 5.3571x over previous
import functools

import jax
import jax.numpy as jnp
from jax import lax
from jax.experimental import pallas as pl
from jax.experimental.pallas import tpu as pltpu
from jax.experimental.pallas import tpu_sc as plsc

L = 16


def kernel(nodes, neighbors, num_sample, features):
    B, S = neighbors.shape
    N, D = features.shape
    KD = D // L

    info = plsc.get_sparse_core_info()
    NC, NS = info.num_cores, info.num_subcores
    NW = NC * NS

    G = 128 // S
    CH = G * S

    rows_per_w = -(-B // NW)
    rows_per_w = -(-rows_per_w // (4 * G)) * (4 * G)
    nchunk = rows_per_w // G
    b_pad = rows_per_w * NW

    nb = neighbors.astype(jnp.int32)
    if b_pad != B:
        nb = jnp.concatenate(
            [nb, jnp.zeros((b_pad - B, S), jnp.int32)], axis=0)
    nb = nb.reshape(NW, nchunk, CH)

    n_stage = -(-(-(-N // NS)) // 8) * 8
    n_pad = n_stage * NS

    mesh = plsc.VectorSubcoreMesh(core_axis_name="c", subcore_axis_name="s")

    @functools.partial(
        pl.kernel,
        mesh=mesh,
        out_type=jax.ShapeDtypeStruct((b_pad, D), jnp.float32),
        scratch_types=[
            pltpu.VMEM_SHARED((n_pad, D), jnp.float32),
            pltpu.VMEM((nchunk, CH), jnp.int32),
            pltpu.VMEM((CH, D), jnp.float32),
            pltpu.VMEM((CH, D), jnp.float32),
            pltpu.VMEM((2 * G, D), jnp.float32),
            pltpu.VMEM((2 * G, D), jnp.float32),
            pltpu.SemaphoreType.DMA,
            pltpu.SemaphoreType.DMA,
            pltpu.SemaphoreType.DMA,
            pltpu.SemaphoreType.DMA,
        ],
    )
    def sc_kernel(nb_hbm, feat_hbm, out_hbm, tab_sh, idx_v, rows0, rows1,
                  obuf0, obuf1, sem0, sem1, wsem0, wsem1):
        sid = lax.axis_index("s")
        wid = sid * NC + lax.axis_index("c")
        pltpu.sync_copy(feat_hbm.at[pl.ds(sid * n_stage, n_stage)],
                        tab_sh.at[pl.ds(sid * n_stage, n_stage)])
        pltpu.sync_copy(nb_hbm.at[wid], idx_v)
        plsc.subcore_barrier()

        def start(g, rows, sem):
            pltpu.async_copy(tab_sh.at[idx_v.at[g]], rows, sem)

        def wait_g(rows, sem):
            pltpu.make_async_copy(tab_sh.at[idx_v.at[0]], rows, sem).wait()

        base = wid * rows_per_w

        def wait_w(obuf, wsem):
            pltpu.make_async_copy(
                obuf, out_hbm.at[pl.ds(base, 2 * G)], wsem).wait()

        def compute(rows_ref, obuf, half):
            def do_row(r, carry):
                b0 = r * S
                accs = tuple(rows_ref[b0, pl.ds(k * L, L)] for k in range(KD))

                def s_body(s, accs):
                    return tuple(
                        jnp.maximum(a, rows_ref[b0 + s, pl.ds(k * L, L)])
                        for k, a in enumerate(accs))

                accs = lax.fori_loop(1, S, s_body, accs)
                for k in range(KD):
                    obuf[half * G + r, pl.ds(k * L, L)] = accs[k]
                return carry

            lax.fori_loop(0, G, do_row, 0)

        start(0, rows0, sem0)
        start(1, rows1, sem1)

        def body(i, carry):
            g = 4 * i
            wait_g(rows0, sem0)

            @pl.when(i > 0)
            def _():
                wait_w(obuf0, wsem0)

            compute(rows0, obuf0, 0)
            start(g + 2, rows0, sem0)

            wait_g(rows1, sem1)
            compute(rows1, obuf0, 1)
            pltpu.async_copy(
                obuf0, out_hbm.at[pl.ds(base + g * G, 2 * G)], wsem0)

            @pl.when(g + 3 < nchunk)
            def _():
                start(g + 3, rows1, sem1)

            wait_g(rows0, sem0)

            @pl.when(i > 0)
            def _():
                wait_w(obuf1, wsem1)

            compute(rows0, obuf1, 0)

            @pl.when(g + 4 < nchunk)
            def _():
                start(g + 4, rows0, sem0)

            wait_g(rows1, sem1)
            compute(rows1, obuf1, 1)
            pltpu.async_copy(
                obuf1, out_hbm.at[pl.ds(base + (g + 2) * G, 2 * G)], wsem1)

            @pl.when(g + 5 < nchunk)
            def _():
                start(g + 5, rows1, sem1)

            return carry

        lax.fori_loop(0, nchunk // 4, body, 0)
        wait_w(obuf0, wsem0)
        wait_w(obuf1, wsem1)

    feat = features
    if n_pad != N:
        feat = jnp.concatenate(
            [feat, jnp.zeros((n_pad - N, D), jnp.float32)], axis=0)
    out = sc_kernel(nb, feat)
    return out[:B]


# speedup vs baseline: 8.4737x; 1.1406x over previous
import functools

import jax
import jax.numpy as jnp
from jax import lax
from jax.experimental import pallas as pl
from jax.experimental.pallas import tpu as pltpu
from jax.experimental.pallas import tpu_sc as plsc

L = 16


def kernel(nodes, neighbors, num_sample, features):
    B, S = neighbors.shape
    N, D = features.shape
    KD = D // L

    info = plsc.get_sparse_core_info()
    NC, NS = info.num_cores, info.num_subcores
    NW = NC * NS

    G = 128 // S
    CH = G * S

    rows_per_w = -(-B // NW)
    rows_per_w = -(-rows_per_w // (4 * G)) * (4 * G)
    nchunk = rows_per_w // G
    nidx = rows_per_w * S

    nb = neighbors.astype(jnp.int32).reshape(B * S)

    feat = features
    n_rows = N
    if n_rows % 8 != 0:
        pad = 8 - n_rows % 8
        feat = jnp.concatenate(
            [feat, jnp.zeros((pad, D), jnp.float32)], axis=0)
        n_rows += pad
    n_stage = -(-(-(-n_rows // NS)) // 8) * 8

    mesh = plsc.VectorSubcoreMesh(core_axis_name="c", subcore_axis_name="s")

    @functools.partial(
        pl.kernel,
        mesh=mesh,
        out_type=jax.ShapeDtypeStruct((B, D), jnp.float32),
        scratch_types=[
            pltpu.VMEM_SHARED((n_rows, D), jnp.float32),
            pltpu.VMEM((nidx,), jnp.int32),
            pltpu.VMEM((CH, D), jnp.float32),
            pltpu.VMEM((CH, D), jnp.float32),
            pltpu.VMEM((2 * G, D), jnp.float32),
            pltpu.VMEM((2 * G, D), jnp.float32),
            pltpu.SemaphoreType.DMA,
            pltpu.SemaphoreType.DMA,
            pltpu.SemaphoreType.DMA,
            pltpu.SemaphoreType.DMA,
        ],
    )
    def sc_kernel(nb_hbm, feat_hbm, out_hbm, tab_sh, idx_v, rows0, rows1,
                  obuf0, obuf1, sem0, sem1, wsem0, wsem1):
        sid = lax.axis_index("s")
        wid = sid * NC + lax.axis_index("c")
        soff = jnp.minimum(sid * n_stage, n_rows - n_stage)
        pltpu.sync_copy(feat_hbm.at[pl.ds(soff, n_stage)],
                        tab_sh.at[pl.ds(soff, n_stage)])
        base = jnp.minimum(wid * rows_per_w, B - rows_per_w)
        pltpu.sync_copy(nb_hbm.at[pl.ds(base * S, nidx)], idx_v)
        plsc.subcore_barrier()

        def start(g, rows, sem):
            pltpu.async_copy(
                tab_sh.at[idx_v.at[pl.ds(g * CH, CH)]], rows, sem)

        def wait_g(rows, sem):
            pltpu.make_async_copy(
                tab_sh.at[idx_v.at[pl.ds(0, CH)]], rows, sem).wait()

        def wait_w(obuf, wsem):
            pltpu.make_async_copy(
                obuf, out_hbm.at[pl.ds(base, 2 * G)], wsem).wait()

        def compute(rows_ref, obuf, half):
            def do_row(r, carry):
                b0 = r * S
                accs = tuple(rows_ref[b0, pl.ds(k * L, L)] for k in range(KD))

                def s_body(s, accs):
                    return tuple(
                        jnp.maximum(a, rows_ref[b0 + s, pl.ds(k * L, L)])
                        for k, a in enumerate(accs))

                accs = lax.fori_loop(1, S, s_body, accs)
                for k in range(KD):
                    obuf[half * G + r, pl.ds(k * L, L)] = accs[k]
                return carry

            lax.fori_loop(0, G, do_row, 0)

        start(0, rows0, sem0)
        start(1, rows1, sem1)

        def body(i, carry):
            g = 4 * i
            wait_g(rows0, sem0)

            @pl.when(i > 0)
            def _():
                wait_w(obuf0, wsem0)

            compute(rows0, obuf0, 0)
            start(g + 2, rows0, sem0)

            wait_g(rows1, sem1)
            compute(rows1, obuf0, 1)
            pltpu.async_copy(
                obuf0, out_hbm.at[pl.ds(base + g * G, 2 * G)], wsem0)
            start(g + 3, rows1, sem1)

            wait_g(rows0, sem0)

            @pl.when(i > 0)
            def _():
                wait_w(obuf1, wsem1)

            compute(rows0, obuf1, 0)

            @pl.when(g + 4 < nchunk)
            def _():
                start(g + 4, rows0, sem0)

            wait_g(rows1, sem1)
            compute(rows1, obuf1, 1)
            pltpu.async_copy(
                obuf1, out_hbm.at[pl.ds(base + (g + 2) * G, 2 * G)], wsem1)

            @pl.when(g + 5 < nchunk)
            def _():
                start(g + 5, rows1, sem1)

            return carry

        lax.fori_loop(0, nchunk // 4, body, 0)
        wait_w(obuf0, wsem0)
        wait_w(obuf1, wsem1)

    return sc_kernel(nb, feat)
